# Initial kernel scaffold; baseline (speedup 1.0000x reference)
import jax
import jax.numpy as jnp
from jax.experimental import pallas as pl


def kernel(x, cluster_centers):
    raise NotImplementedError("write your pallas kernel here")


# speedup vs baseline: 3.9884x; 3.9884x over previous
import jax
import jax.numpy as jnp
from jax.experimental import pallas as pl

_K = 512
_BINS = _K + 1
_BS = 4


def _body(x_ref, c_ref, o_ref):
    xf = x_ref[...]
    cm = c_ref[...]
    cn = jnp.sum(cm * cm, axis=0, keepdims=True)
    prod = jax.lax.dot_general(
        xf, cm, (((1,), (0,)), ((), ())),
        precision=jax.lax.Precision.HIGHEST,
        preferred_element_type=jnp.float32,
    )
    s = cn - 2.0 * prod
    m = jnp.min(s, axis=1, keepdims=True)
    ki = jax.lax.broadcasted_iota(jnp.int32, s.shape, 1)
    code = jnp.min(jnp.where(s == m, ki, _K), axis=1, keepdims=True)
    bins = jax.lax.broadcasted_iota(jnp.int32, (1024, _BINS), 1)
    oh = (bins == code + 1).astype(jnp.float32)
    o_ref[...] = jnp.sum(oh.reshape(64, 16, _BINS), axis=1) * (1.0 / 16.0)


def kernel(x, cluster_centers):
    B, C, H, W = x.shape
    nh, nw = H // _BS, W // _BS
    xb = (
        x.transpose(0, 2, 3, 1)
        .reshape(B, nh, _BS, nw, _BS, C)
        .transpose(0, 1, 3, 2, 4, 5)
        .reshape(B * H * W, C)
    )
    cm = cluster_centers.reshape(_K, C).T
    hist = pl.pallas_call(
        _body,
        out_shape=jax.ShapeDtypeStruct((B * nh * nw, _BINS), jnp.float32),
    )(xb, cm)
    return hist.reshape(B, nh * nw, _BINS)
